# Initial kernel scaffold; baseline (speedup 1.0000x reference)
import jax
import jax.numpy as jnp
from jax.experimental import pallas as pl


def kernel(constraint_features_s, edge_index_s, edge_attr_s, variable_features_s, variable_features_s_batch, constraint_features_t, edge_index_t, edge_attr_t, variable_features_t, variable_features_t_batch, Wl, bl, Wr, W1, b1, W2):
    raise NotImplementedError("write your pallas kernel here")


# speedup vs baseline: 50.6349x; 50.6349x over previous
import functools

import jax
import jax.numpy as jnp
import numpy as np
from jax import lax
from jax.experimental import pallas as pl
from jax.experimental.pallas import tpu as pltpu
from jax.experimental.pallas import tpu_sc as plsc

N = 50000
E = 1600000
B = 64
EMB = 16
K = 16
NCONV = 4

NT = 16
SL = 3200
NPAD = NT * SL
CHK = 2000
EPT = E // NT
NEG = jnp.float32(-3.0e38)
QNP = NPAD // 4
HB = 128
BLK = 2048

_sc_params = pltpu.CompilerParams(needs_layout_passes=False)
_mesh = plsc.VectorSubcoreMesh(core_axis_name="c", subcore_axis_name="s")


def _extract_i32(vec, lane):
    return jnp.max(jnp.where(lax.iota(jnp.int32, 16) == lane, vec, 0))


def _extract_f32(vec, lane):
    return jnp.sum(jnp.where(lax.iota(jnp.int32, 16) == lane, vec,
                             jnp.float32(0.0)))


def _round_bf16(v):
    b = plsc.bitcast(v, jnp.int32)
    r = (b + 0x7FFF + ((b >> 16) & 1)) & jnp.int32(-65536)
    return plsc.bitcast(r, jnp.float32)


@functools.partial(
    pl.kernel,
    out_type=(
        jax.ShapeDtypeStruct((2 * NPAD,), jnp.float32),
        jax.ShapeDtypeStruct((8 * NPAD + BLK,), jnp.float32),
        jax.ShapeDtypeStruct((2, HB), jnp.int32),
    ),
    mesh=_mesh,
    compiler_params=_sc_params,
    scratch_types=[
        pltpu.VMEM((NPAD,), jnp.float32),
        pltpu.VMEM((NPAD,), jnp.float32),
        pltpu.VMEM((CHK,), jnp.int32),
        pltpu.VMEM((CHK,), jnp.int32),
        pltpu.VMEM((HB,), jnp.int32),
        pltpu.VMEM((NT * HB,), jnp.int32),
        pltpu.VMEM((32,), jnp.float32),
        pltpu.VMEM_SHARED((NT * QNP,), jnp.float32),
        pltpu.VMEM_SHARED((NT * HB,), jnp.int32),
    ],
)
def _edge_agg(cons_hbm, src_hbm, dst_hbm, vb_hbm, varT_hbm, wsc_hbm,
              mean_hbm, keys_hbm, cnt_hbm,
              cons_t, acc_t, sbuf, dbuf, hist_t, hred_t, wbuf,
              part_sh, hist_sh):
    tid = lax.axis_index("s")
    side = lax.axis_index("c")

    pltpu.sync_copy(cons_hbm.at[pl.ds(side * N, N)], cons_t.at[pl.ds(0, N)])
    pltpu.sync_copy(wsc_hbm, wbuf)

    zf = jnp.zeros((16,), jnp.float32)
    ones = jnp.ones((16,), jnp.float32)

    def zero_acc(i, _):
        acc_t[pl.ds(i * 16, 16)] = zf
        return 0

    lax.fori_loop(0, NPAD // 16, zero_acc, 0)

    ebase = side * E + tid * EPT

    def chunk1(c, _):
        off = ebase + c * CHK
        pltpu.sync_copy(src_hbm.at[pl.ds(off, CHK)], sbuf)
        pltpu.sync_copy(dst_hbm.at[pl.ds(off, CHK)], dbuf)

        def inner(j, _):
            si = sbuf[pl.ds(j * 16, 16)]
            di = dbuf[pl.ds(j * 16, 16)]
            vals = plsc.load_gather(cons_t, [si])
            plsc.addupdate_scatter(acc_t, [di], vals)
            return 0

        lax.fori_loop(0, CHK // 16, inner, 0)
        return 0

    lax.fori_loop(0, EPT // CHK, chunk1, 0)

    me = tid * SL
    moff = (tid % 4) * SL
    bufT = cons_t.at[pl.ds(0, SL)]
    bufA = cons_t.at[pl.ds(SL, SL)]
    bufD = cons_t.at[pl.ds(2 * SL, SL)]
    bufM = cons_t.at[pl.ds(3 * SL, SL)]
    bufS = cons_t.at[pl.ds(4 * SL, SL)]
    vcs = [acc_t.at[pl.ds(j * SL, SL)] for j in range(6)]

    def zero_sl(i, _):
        bufA[pl.ds(i * 16, 16)] = zf
        bufD[pl.ds(i * 16, 16)] = zf
        return 0

    lax.fori_loop(0, SL // 16, zero_sl, 0)

    def reduce_quarters(dst_buf):
        for q in range(4):
            pltpu.sync_copy(
                acc_t.at[pl.ds(q * QNP, QNP)],
                part_sh.at[pl.ds(pl.multiple_of(tid * QNP, 128), QNP)])
            plsc.subcore_barrier()

            @pl.when(tid // 4 == q)
            def _():
                for j in range(NT):
                    pltpu.sync_copy(
                        part_sh.at[
                            pl.ds(pl.multiple_of(j * QNP + moff, 128), SL)],
                        bufT)

                    def addpart(i, _):
                        dst_buf[pl.ds(i * 16, 16)] = (
                            dst_buf[pl.ds(i * 16, 16)]
                            + bufT[pl.ds(i * 16, 16)])
                        return 0

                    lax.fori_loop(0, SL // 16, addpart, 0)

            plsc.subcore_barrier()

    reduce_quarters(bufA)

    lax.fori_loop(0, NPAD // 16, zero_acc, 0)

    def chunk2(c, _):
        off = ebase + c * CHK
        pltpu.sync_copy(dst_hbm.at[pl.ds(off, CHK)], dbuf)

        def inner(j, _):
            di = dbuf[pl.ds(j * 16, 16)]
            plsc.addupdate_scatter(acc_t, [di], ones)
            return 0

        lax.fori_loop(0, CHK // 16, inner, 0)
        return 0

    lax.fori_loop(0, EPT // CHK, chunk2, 0)
    reduce_quarters(bufD)

    zi = jnp.zeros((16,), jnp.int32)
    onesi = jnp.ones((16,), jnp.int32)
    for hc in range(HB // 16):
        hist_t[pl.ds(hc * 16, 16)] = zi

    def histchunk(c, _):
        pltpu.sync_copy(
            vb_hbm.at[pl.ds(side * NPAD + tid * SL + c * CHK, CHK)], dbuf)
        jmax = jnp.minimum(CHK, SL - c * CHK) // 16

        def inner(j, _):
            bi = dbuf[pl.ds(j * 16, 16)]
            plsc.addupdate_scatter(hist_t, [bi], onesi)
            return 0

        lax.fori_loop(0, jmax, inner, 0)
        return 0

    lax.fori_loop(0, (SL + CHK - 1) // CHK, histchunk, 0)
    pltpu.sync_copy(
        hist_t, hist_sh.at[pl.ds(pl.multiple_of(tid * HB, 128), HB)])

    plsc.subcore_barrier()

    @pl.when(tid == 0)
    def _():
        pltpu.sync_copy(hist_sh, hred_t)
        for hc in range(HB // 16):
            acc = jnp.zeros((16,), jnp.int32)
            for j in range(NT):
                acc = acc + hred_t[pl.ds(j * HB + hc * 16, 16)]
            hist_t[pl.ds(hc * 16, 16)] = acc
        pltpu.sync_copy(hist_t, cnt_hbm.at[side])

    def meanchunk(i, _):
        a = bufA[pl.ds(i * 16, 16)]
        d = bufD[pl.ds(i * 16, 16)]
        bufM[pl.ds(i * 16, 16)] = a / jnp.maximum(d, jnp.float32(1.0))
        return 0

    lax.fori_loop(0, SL // 16, meanchunk, 0)
    pltpu.sync_copy(bufM, mean_hbm.at[pl.ds(side * NPAD + me, SL)])

    for j in range(6):
        pltpu.sync_copy(varT_hbm.at[side, j, pl.ds(me, SL)], vcs[j])

    w0 = wbuf[pl.ds(0, 16)]
    w1 = wbuf[pl.ds(16, 16)]

    def wscal(i, k):
        lane = i * 8 + k
        v = w0 if lane < 16 else w1
        return _extract_f32(v, lane % 16)

    for i in range(NCONV):
        wl = wscal(i, 0)
        wrs = [wscal(i, 1 + j) for j in range(6)]
        bl = wscal(i, 7)

        def keychunk(c, _):
            s = bufM[pl.ds(c * 16, 16)] * wl + bl
            for j in range(6):
                s = s + _round_bf16(vcs[j][pl.ds(c * 16, 16)]) * wrs[j]
            bufS[pl.ds(c * 16, 16)] = s
            return 0

        lax.fori_loop(0, SL // 16, keychunk, 0)
        pltpu.sync_copy(
            bufS, keys_hbm.at[pl.ds((side * 4 + i) * NPAD + me, SL)])


@functools.partial(
    pl.kernel,
    out_type=jax.ShapeDtypeStruct((512, 16, 16), jnp.float32),
    mesh=_mesh,
    compiler_params=_sc_params,
    scratch_types=[
        pltpu.VMEM((BLK,), jnp.float32),
        pltpu.VMEM((128,), jnp.int32),
        pltpu.VMEM((16,), jnp.int32),
        pltpu.VMEM((16, 128), jnp.float32),
        pltpu.VMEM((16, 16), jnp.float32),
        pltpu.SemaphoreType.DMA,
    ],
)
def _topk_gather(keys_hbm, starts_hbm, nodefeat_hbm,
                 self_hbm,
                 sbuf, stv, idxv, fsel, csel, sem0):
    wid = lax.axis_index("s") * 2 + lax.axis_index("c")
    t0 = wid * 16
    side = t0 // 256
    conv = (t0 // 64) % 4
    kbase = (side * 4 + conv) * NPAD

    pltpu.sync_copy(starts_hbm.at[side], stv)

    lane = lax.iota(jnp.int32, 16)

    def getstart(b):
        chunk = stv[pl.ds((b // 16) * 16, 16)]
        return _extract_i32(chunk, b % 16)

    for tl in range(16):
        t = t0 + tl
        b = t % 64
        st = getstart(b)
        en = getstart(b + 1)
        blk0 = (st // 16) * 16
        nblk = (en - blk0 + BLK - 1) // BLK

        def blkbody(blk, carry):
            runk, runv = carry
            boff = blk0 + blk * BLK
            pltpu.sync_copy(keys_hbm.at[pl.ds(kbase + boff, BLK)], sbuf)
            jmax = jnp.minimum((en - boff + 15) // 16, BLK // 16)

            def chunkbody(j, carry2):
                rk, rv = carry2
                k = sbuf[pl.ds(j * 16, 16)]
                gidx = boff + j * 16 + lane
                valid = (gidx >= st) & (gidx < en)
                kk = jnp.where(valid, k, NEG)
                ck, cv = plsc.sort_key_val(kk, gidx, descending=True)
                rck = lax.rev(ck, (0,))
                rcv = lax.rev(cv, (0,))
                m = rk >= rck
                mk = jnp.where(m, rk, rck)
                mv = jnp.where(m, rv, rcv)
                r = plsc.sort_key_val(mk, mv, descending=True)
                return (r[0], r[1])

            return lax.fori_loop(0, jmax, chunkbody, (runk, runv))

        runk, runv = lax.fori_loop(
            0, nblk, blkbody,
            (jnp.full((16,), NEG, jnp.float32), jnp.zeros((16,), jnp.int32)))

        idxv[...] = side * N + runv
        pltpu.async_copy(nodefeat_hbm.at[idxv], fsel, sem0).wait()
        for r in range(16):
            csel[r, pl.ds(0, 16)] = fsel[r, pl.ds(0, 16)]
        pltpu.sync_copy(csel, self_hbm.at[t])


def _dense_body(sf_ref, cnt_ref, wt_ref, blt_ref, w1_ref, b1_ref, w2_ref,
                out_ref):
    cnt = cnt_ref[...]
    rank_iota = lax.broadcasted_iota(jnp.int32, (64, 256), 1) // 16
    scores = []
    for s in range(2):
        cnt_s = cnt[s].reshape(64, 1)
        maskf = (rank_iota < cnt_s).astype(jnp.float32)
        hid = jnp.zeros((64, 256), jnp.float32)
        for i in range(NCONV):
            r0 = (s * 4 + i) * 64
            feat_si = sf_ref[pl.ds(r0, 64), :]
            h_si = jnp.dot(feat_si, wt_ref[i],
                           preferred_element_type=jnp.float32)
            h_si = (h_si + blt_ref[i].reshape(1, 256)) * maskf
            hid = hid + jnp.dot(h_si, w1_ref[pl.ds(i * 256, 256), :],
                                preferred_element_type=jnp.float32)
        hid = jnp.maximum(hid + b1_ref[...].reshape(1, 256), 0.0)
        scores.append(jnp.dot(hid, w2_ref[...],
                              preferred_element_type=jnp.float32))
    d = (scores[0] - scores[1]).reshape(64)
    out_ref[...] = 1.0 / (1.0 + jnp.exp(-d))


def kernel(constraint_features_s, edge_index_s, edge_attr_s,
           variable_features_s, variable_features_s_batch,
           constraint_features_t, edge_index_t, edge_attr_t,
           variable_features_t, variable_features_t_batch,
           Wl, bl, Wr, W1, b1, W2):
    f32 = jnp.float32

    cons_all = jnp.concatenate([constraint_features_s[:, 0],
                                constraint_features_t[:, 0]])
    src_all = jnp.concatenate([edge_index_s[1], edge_index_t[1]])
    dst_all = jnp.concatenate([edge_index_s[0], edge_index_t[0]])
    pad_b = jnp.full((NPAD - N,), B, jnp.int32)
    pad_b2 = jnp.full((NPAD - N + CHK,), B, jnp.int32)
    vb_all = jnp.concatenate([variable_features_s_batch, pad_b,
                              variable_features_t_batch, pad_b2])
    varT_all = jnp.stack([
        jnp.pad(variable_features_s.T, ((0, 0), (0, NPAD - N))),
        jnp.pad(variable_features_t.T, ((0, 0), (0, NPAD - N))),
    ])
    var_all = jnp.concatenate([variable_features_s, variable_features_t])

    wr15 = Wr[:, :, 15].reshape(4, 6).astype(jnp.bfloat16).astype(f32)
    wsc = jnp.concatenate(
        [Wl[:, 0, 15:16], wr15, bl[:, 15:16]],
        axis=1).reshape(32).astype(f32)

    mean_all, keys_all, cnt2 = _edge_agg(
        cons_all, src_all, dst_all, vb_all, varT_all, wsc)

    cnt = cnt2[:, :B]
    starts = jnp.concatenate(
        [jnp.zeros((2, 1), jnp.int32), jnp.cumsum(cnt, axis=1)], axis=1)
    starts_pad = jnp.pad(starts, ((0, 0), (0, 128 - 65)))

    mean2 = mean_all.reshape(2, NPAD)[:, :N].reshape(2 * N, 1)
    nodefeat = jnp.pad(
        jnp.concatenate([var_all, jnp.zeros((2 * N, 1), f32), mean2],
                        axis=1),
        ((0, 0), (0, 120)))

    self_rows = _topk_gather(keys_all, starts_pad, nodefeat)
    sf = self_rows.reshape(512, 256)

    wcomb = jnp.concatenate(
        [Wr, jnp.zeros((NCONV, 1, EMB), f32), Wl,
         jnp.zeros((NCONV, 8, EMB), f32)], axis=1)
    eye = jnp.eye(16, dtype=f32)
    wtilde = jnp.einsum('rq,ikd->irkqd', eye, wcomb).reshape(4, 256, 256)
    bltile = jnp.tile(bl, (1, K))

    out = pl.pallas_call(
        _dense_body,
        out_shape=jax.ShapeDtypeStruct((B,), f32),
    )(sf, cnt, wtilde, bltile, W1, b1, W2)
    return out
